# baseline (speedup 1.0000x reference)
import math

import jax
import jax.numpy as jnp
from jax import lax
from jax.experimental import pallas as pl
from jax.experimental.pallas import tpu as pltpu
from jax.experimental.pallas import tpu_sc as plsc

D_MODEL = 64
LANES = 16
NUM_CORES = 2
NUM_WORKERS = 32
ABLK = 128
NBUF = 3
SCALE = math.sqrt(D_MODEL)


def _emb_body(xt_hbm, lut_hbm, out_hbm, idx_t, pbuf, gbuf, tbuf, gsem, ssem):
    nb = xt_hbm.shape[0]
    wid = lax.axis_index("s") * NUM_CORES + lax.axis_index("c")
    a0 = wid * ABLK

    pltpu.sync_copy(xt_hbm.at[:, pl.ds(a0, ABLK)], idx_t)

    def compute_pairs(b, slot):
        for g in range(ABLK // LANES):
            pbuf[slot, pl.ds(g * LANES, LANES)] = (
                idx_t[b, pl.ds(g * LANES, LANES)] >> 1)

    def start_gather(b, slot):
        compute_pairs(b, slot)
        pltpu.async_copy(lut_hbm.at[pbuf.at[slot]], gbuf.at[slot],
                         gsem.at[slot])

    for s in range(NBUF):
        start_gather(s, s)

    def slab(b, _):
        for s in range(NBUF):
            g = b * NBUF + s
            pltpu.make_async_copy(
                lut_hbm.at[pbuf.at[s]], gbuf.at[s], gsem.at[s]).wait()

            @pl.when(b > 0)
            def _wait_scatter():
                pltpu.make_async_copy(
                    tbuf.at[s], out_hbm.at[0, :, wid], ssem.at[s]).wait()

            def col(i0, _):
                rows = lax.iota(jnp.int32, LANES) + i0
                h = jnp.bitwise_and(idx_t[g, pl.ds(i0, LANES)], 1) * D_MODEL
                for c in range(D_MODEL):
                    v = plsc.load_gather(gbuf.at[s], [rows, h + c])
                    tbuf[s, c // 8, c % 8, pl.ds(i0, LANES)] = v * SCALE
                return 0

            lax.fori_loop(0, ABLK // LANES, lambda k, u: col(k * LANES, u),
                          0, unroll=False)

            pltpu.async_copy(tbuf.at[s], out_hbm.at[g, :, wid], ssem.at[s])

            @pl.when(g + NBUF < nb)
            def _next():
                start_gather(g + NBUF, s)
        return 0

    lax.fori_loop(0, nb // NBUF, slab, 0)
    tail0 = (nb // NBUF) * NBUF
    for g in range(tail0, nb):
        s = g % NBUF
        pltpu.make_async_copy(
            lut_hbm.at[pbuf.at[s]], gbuf.at[s], gsem.at[s]).wait()
        pltpu.make_async_copy(
            tbuf.at[s], out_hbm.at[0, :, wid], ssem.at[s]).wait()

        def col2(i0, _, g=g, s=s):
            rows = lax.iota(jnp.int32, LANES) + i0
            h = jnp.bitwise_and(idx_t[g, pl.ds(i0, LANES)], 1) * D_MODEL
            for c in range(D_MODEL):
                v = plsc.load_gather(gbuf.at[s], [rows, h + c])
                tbuf[s, c // 8, c % 8, pl.ds(i0, LANES)] = v * SCALE
            return 0

        lax.fori_loop(0, ABLK // LANES, lambda k, u: col2(k * LANES, u), 0,
                      unroll=False)
        pltpu.async_copy(tbuf.at[s], out_hbm.at[g, :, wid], ssem.at[s])

    for s in range(NBUF):
        pltpu.make_async_copy(
            tbuf.at[s], out_hbm.at[0, :, wid], ssem.at[s]).wait()


def kernel(x, lut):
    n_rows, n_cols = x.shape
    xt = jnp.swapaxes(x, 0, 1).astype(jnp.int32)
    lut_pairs = lut.reshape(lut.shape[0] // 2, 2 * D_MODEL)

    out5 = pl.kernel(
        _emb_body,
        out_type=jax.ShapeDtypeStruct(
            (n_cols, D_MODEL // 8, NUM_WORKERS, 8, ABLK), jnp.float32),
        mesh=plsc.VectorSubcoreMesh(core_axis_name="c", subcore_axis_name="s"),
        compiler_params=pltpu.CompilerParams(
            use_tc_tiling_on_sc=True, needs_layout_passes=False),
        scratch_types=[
            pltpu.VMEM((n_cols, ABLK), jnp.int32),
            pltpu.VMEM((NBUF, ABLK), jnp.int32),
            pltpu.VMEM((NBUF, ABLK, 2 * D_MODEL), jnp.float32),
            pltpu.VMEM((NBUF, D_MODEL // 8, 8, ABLK), jnp.float32),
            pltpu.SemaphoreType.DMA((NBUF,)),
            pltpu.SemaphoreType.DMA((NBUF,)),
        ],
    )(xt, lut_pairs)

    return out5.transpose(2, 4, 0, 1, 3).reshape(n_rows, n_cols, D_MODEL)


# speedup vs baseline: 1.0277x; 1.0277x over previous
import math

import jax
import jax.numpy as jnp
from jax import lax
from jax.experimental import pallas as pl
from jax.experimental.pallas import tpu as pltpu
from jax.experimental.pallas import tpu_sc as plsc

D_MODEL = 64
LANES = 16
NUM_CORES = 2
NUM_WORKERS = 32
ABLK = 128
NBUF = 3
TPAD = 129
SCALE = math.sqrt(D_MODEL)


def _emb_body(xt_hbm, lut_hbm, out_hbm, idx_t, pbuf, gbuf, tbuf, gsem, ssem):
    nb = xt_hbm.shape[0]
    wid = lax.axis_index("s") * NUM_CORES + lax.axis_index("c")
    a0 = wid * ABLK

    pltpu.sync_copy(xt_hbm.at[:, pl.ds(a0, ABLK)], idx_t)

    def compute_pairs(b, slot):
        for g in range(ABLK // LANES):
            pbuf[slot, pl.ds(g * LANES, LANES)] = (
                idx_t[b, pl.ds(g * LANES, LANES)] >> 1)

    def start_gather(b, slot):
        compute_pairs(b, slot)
        pltpu.async_copy(lut_hbm.at[pbuf.at[slot]], gbuf.at[slot],
                         gsem.at[slot])

    for s in range(NBUF):
        start_gather(s, s)

    lanes_i = lax.iota(jnp.int32, LANES)

    def transpose_rows(g, s):
        def row(i, _):
            hb = plsc.load_gather(
                idx_t, [jnp.full((LANES,), g, jnp.int32),
                        jnp.full((LANES,), i, jnp.int32)])
            h = jnp.bitwise_and(hb, 1) * D_MODEL
            rvec = jnp.full((LANES,), i, jnp.int32)
            for j in range(D_MODEL // LANES):
                cols = h + j * LANES + lanes_i
                v = plsc.load_gather(gbuf.at[s], [rvec, cols])
                c8 = jnp.right_shift(j * LANES + lanes_i, 3)
                cr = jnp.bitwise_and(j * LANES + lanes_i, 7)
                plsc.store_scatter(tbuf.at[s], [c8, cr, rvec], v * SCALE)
            return 0

        lax.fori_loop(0, ABLK, row, 0, unroll=4)

    def slab(b, _):
        for s in range(NBUF):
            g = b * NBUF + s
            pltpu.make_async_copy(
                lut_hbm.at[pbuf.at[s]], gbuf.at[s], gsem.at[s]).wait()

            @pl.when(b > 0)
            def _wait_scatter():
                pltpu.make_async_copy(
                    tbuf.at[s, :, :, pl.ds(0, ABLK)],
                    out_hbm.at[0, :, wid], ssem.at[s]).wait()

            transpose_rows(g, s)

            pltpu.async_copy(tbuf.at[s, :, :, pl.ds(0, ABLK)],
                             out_hbm.at[g, :, wid], ssem.at[s])

            @pl.when(g + NBUF < nb)
            def _next():
                start_gather(g + NBUF, s)
        return 0

    lax.fori_loop(0, nb // NBUF, slab, 0)
    tail0 = (nb // NBUF) * NBUF
    for g in range(tail0, nb):
        s = g % NBUF
        pltpu.make_async_copy(
            lut_hbm.at[pbuf.at[s]], gbuf.at[s], gsem.at[s]).wait()
        pltpu.make_async_copy(
            tbuf.at[s, :, :, pl.ds(0, ABLK)], out_hbm.at[0, :, wid],
            ssem.at[s]).wait()
        transpose_rows(g, s)
        pltpu.async_copy(tbuf.at[s, :, :, pl.ds(0, ABLK)],
                         out_hbm.at[g, :, wid], ssem.at[s])

    for s in range(NBUF):
        pltpu.make_async_copy(
            tbuf.at[s, :, :, pl.ds(0, ABLK)], out_hbm.at[0, :, wid],
            ssem.at[s]).wait()


def kernel(x, lut):
    n_rows, n_cols = x.shape
    xt = jnp.swapaxes(x, 0, 1).astype(jnp.int32)
    lut_pairs = lut.reshape(lut.shape[0] // 2, 2 * D_MODEL)

    out5 = pl.kernel(
        _emb_body,
        out_type=jax.ShapeDtypeStruct(
            (n_cols, D_MODEL // 8, NUM_WORKERS, 8, ABLK), jnp.float32),
        mesh=plsc.VectorSubcoreMesh(core_axis_name="c", subcore_axis_name="s"),
        compiler_params=pltpu.CompilerParams(
            use_tc_tiling_on_sc=True, needs_layout_passes=False),
        scratch_types=[
            pltpu.VMEM((n_cols, ABLK), jnp.int32),
            pltpu.VMEM((NBUF, ABLK), jnp.int32),
            pltpu.VMEM((NBUF, ABLK, 2 * D_MODEL), jnp.float32),
            pltpu.VMEM((NBUF, D_MODEL // 8, 8, TPAD), jnp.float32),
            pltpu.SemaphoreType.DMA((NBUF,)),
            pltpu.SemaphoreType.DMA((NBUF,)),
        ],
    )(xt, lut_pairs)

    return out5.transpose(2, 4, 0, 1, 3).reshape(n_rows, n_cols, D_MODEL)


# speedup vs baseline: 1.5542x; 1.5123x over previous
import math

import jax
import jax.numpy as jnp
from jax import lax
from jax.experimental import pallas as pl
from jax.experimental.pallas import tpu as pltpu
from jax.experimental.pallas import tpu_sc as plsc

D_MODEL = 64
LANES = 16
NUM_CORES = 2
NUM_WORKERS = 32
CHUNK = 128
NBUF = 3
SCALE = math.sqrt(D_MODEL)


def _emb_body(xt_hbm, lut_hbm, out_hbm, idx_t, pbuf, gbuf, sbuf,
              gsem, ssem):
    nb = xt_hbm.shape[0]
    wid = lax.axis_index("s") * NUM_CORES + lax.axis_index("c")
    a0 = wid * CHUNK

    pltpu.sync_copy(xt_hbm.at[:, pl.ds(a0, CHUNK)], idx_t)

    def start_gather(b, slot):
        for g in range(CHUNK // LANES):
            sl = pl.ds(g * LANES, LANES)
            pbuf[slot, sl] = idx_t[b, sl] >> 1
        pltpu.async_copy(lut_hbm.at[pbuf.at[slot]], gbuf.at[slot],
                         gsem.at[slot])

    for s in range(NBUF):
        start_gather(s, s)

    def process(g, s):
        gv = jnp.full((LANES,), g, jnp.int32)

        def row(i, _):
            hb = plsc.load_gather(idx_t, [gv, jnp.full((LANES,), i, jnp.int32)])
            m = jnp.bitwise_and(hb, 1) == 1
            for j in range(D_MODEL // LANES):
                lo = gbuf[s, i, pl.ds(j * LANES, LANES)]
                hi = gbuf[s, i, pl.ds(D_MODEL + j * LANES, LANES)]
                sbuf[s, i, pl.ds(j * LANES, LANES)] = (
                    jnp.where(m, hi, lo) * SCALE)
            return 0

        lax.fori_loop(0, CHUNK, row, 0, unroll=8)
        pltpu.async_copy(sbuf.at[s], out_hbm.at[pl.ds(a0, CHUNK), g, :],
                         ssem.at[s])

    def slab(b, _):
        for s in range(NBUF):
            g = b * NBUF + s
            pltpu.make_async_copy(
                lut_hbm.at[pbuf.at[s]], gbuf.at[s], gsem.at[s]).wait()

            @pl.when(b > 0)
            def _wait_scatter():
                pltpu.make_async_copy(
                    sbuf.at[s], out_hbm.at[pl.ds(a0, CHUNK), 0, :],
                    ssem.at[s]).wait()

            process(g, s)

            @pl.when(g + NBUF < nb)
            def _next():
                start_gather(g + NBUF, s)
        return 0

    lax.fori_loop(0, nb // NBUF, slab, 0)
    tail0 = (nb // NBUF) * NBUF
    for g in range(tail0, nb):
        s = g % NBUF
        pltpu.make_async_copy(
            lut_hbm.at[pbuf.at[s]], gbuf.at[s], gsem.at[s]).wait()
        pltpu.make_async_copy(
            sbuf.at[s], out_hbm.at[pl.ds(a0, CHUNK), 0, :],
            ssem.at[s]).wait()
        process(g, s)

    for s in range(NBUF):
        pltpu.make_async_copy(
            sbuf.at[s], out_hbm.at[pl.ds(a0, CHUNK), 0, :],
            ssem.at[s]).wait()


def kernel(x, lut):
    n_rows, n_cols = x.shape
    xt = jnp.swapaxes(x, 0, 1).astype(jnp.int32)
    lut_pairs = lut.reshape(lut.shape[0] // 2, 2 * D_MODEL)

    return pl.kernel(
        _emb_body,
        out_type=jax.ShapeDtypeStruct((n_rows, n_cols, D_MODEL), jnp.float32),
        mesh=plsc.VectorSubcoreMesh(core_axis_name="c", subcore_axis_name="s"),
        compiler_params=pltpu.CompilerParams(
            use_tc_tiling_on_sc=True, needs_layout_passes=False),
        scratch_types=[
            pltpu.VMEM((n_cols, CHUNK), jnp.int32),
            pltpu.VMEM((NBUF, CHUNK), jnp.int32),
            pltpu.VMEM((NBUF, CHUNK, 2 * D_MODEL), jnp.float32),
            pltpu.VMEM((NBUF, CHUNK, D_MODEL), jnp.float32),
            pltpu.SemaphoreType.DMA((NBUF,)),
            pltpu.SemaphoreType.DMA((NBUF,)),
        ],
    )(xt, lut_pairs)
